# baseline (speedup 1.0000x reference)
import jax
import jax.numpy as jnp
from jax.experimental import pallas as pl

EMBED_DIM = 32
HALF = 16
N_FEATURES = 5
N_TYPES = 6
TB = 6400


def _body(mtf_ref, ft_ref, tt_ref, w1_ref, b1_ref, w2_ref, b2_ref, wf_ref,
          bf_ref, g_ref, beta_ref, out_ref):
    f32 = jnp.float32
    A32 = jnp.full((EMBED_DIM, EMBED_DIM), 1.0 / EMBED_DIM, f32)

    ft = jnp.dot(tt_ref[...], wf_ref[0:HALF, :], preferred_element_type=f32)
    ft = ft + bf_ref[...]
    ft_c = ft - jnp.dot(ft, A32, preferred_element_type=f32)
    ftT = ft_c.T

    wfb = wf_ref[HALF:EMBED_DIM, :]
    wfb_c = wfb - jnp.dot(wfb, A32, preferred_element_type=f32)
    wfbT = wfb_c.T

    w1T = w1_ref[...].T
    w2T = w2_ref[...].T
    b1T = b1_ref[...].T
    b2T = b2_ref[...].T
    gT = g_ref[...].T
    betaT = beta_ref[...].T

    m = mtf_ref[0]
    oh = jnp.concatenate(
        [(m == float(k)).astype(f32) for k in range(N_TYPES)], axis=0)

    fT = ft_ref[...].T
    hT = jnp.dot(w1T, fT, preferred_element_type=f32) + b1T
    hT = 0.5 * hT * (1.0 + jax.lax.erf(hT * 0.7071067811865476))
    featT = jnp.dot(w2T, hT, preferred_element_type=f32) + b2T

    dT = (jnp.dot(wfbT, featT, preferred_element_type=f32)
          + jnp.dot(ftT, oh, preferred_element_type=f32))

    varT = jnp.dot(A32[0:1, :], dT * dT, preferred_element_type=f32)
    yT = dT * jax.lax.rsqrt(varT + 1e-5) * gT + betaT
    y = yT.T
    out_ref[...] = y.reshape(out_ref.shape)


@jax.jit
def kernel(mutation_types, features, type_table, W1, b1, W2, b2, Wf, bf,
           ln_gamma, ln_beta):
    B, M = mutation_types.shape
    N = B * M
    nblk = N // TB
    f2 = features.reshape(N, N_FEATURES)
    mtf = mutation_types.astype(jnp.float32).reshape(nblk, 1, TB)

    small = lambda shp: pl.BlockSpec(shp, lambda i: (0,) * len(shp))
    out = pl.pallas_call(
        _body,
        grid=(nblk,),
        in_specs=[
            pl.BlockSpec((1, 1, TB), lambda i: (i, 0, 0)),
            pl.BlockSpec((TB, N_FEATURES), lambda i: (i, 0)),
            small((N_TYPES, HALF)),
            small((N_FEATURES, HALF)),
            small((1, HALF)),
            small((HALF, HALF)),
            small((1, HALF)),
            small((EMBED_DIM, EMBED_DIM)),
            small((1, EMBED_DIM)),
            small((1, EMBED_DIM)),
            small((1, EMBED_DIM)),
        ],
        out_specs=pl.BlockSpec((TB // M, M, EMBED_DIM), lambda i: (i, 0, 0)),
        out_shape=jax.ShapeDtypeStruct((B, M, EMBED_DIM), jnp.float32),
    )(mtf, f2, type_table, W1, b1.reshape(1, HALF), W2, b2.reshape(1, HALF),
      Wf, bf.reshape(1, EMBED_DIM), ln_gamma.reshape(1, EMBED_DIM),
      ln_beta.reshape(1, EMBED_DIM))
    return out


# speedup vs baseline: 1.0025x; 1.0025x over previous
import jax
import jax.numpy as jnp
from jax.experimental import pallas as pl

EMBED_DIM = 32
HALF = 16
N_FEATURES = 5
N_TYPES = 6
TB = 6400


def _body(mtf_ref, ft_ref, tt_ref, w1_ref, b1_ref, w2_ref, b2_ref, wf_ref,
          bf_ref, g_ref, beta_ref, out_ref):
    f32 = jnp.float32
    A32 = jnp.full((EMBED_DIM, EMBED_DIM), 1.0 / EMBED_DIM, f32)

    ft = jnp.dot(tt_ref[...], wf_ref[0:HALF, :], preferred_element_type=f32)
    ft = ft + bf_ref[...]
    ft_c = ft - jnp.dot(ft, A32, preferred_element_type=f32)
    ftT = ft_c.T

    wfb = wf_ref[HALF:EMBED_DIM, :]
    wfb_c = wfb - jnp.dot(wfb, A32, preferred_element_type=f32)
    wfbT = wfb_c.T

    w1T = w1_ref[...].T
    w2T = w2_ref[...].T
    b1T = b1_ref[...].T
    b2T = b2_ref[...].T
    gT = g_ref[...].T
    betaT = beta_ref[...].T

    m = mtf_ref[0]
    oh = jnp.concatenate(
        [(m == float(k)).astype(f32) for k in range(N_TYPES)], axis=0)

    fT = ft_ref[...].reshape(TB, N_FEATURES).T
    hT = jnp.dot(w1T, fT, preferred_element_type=f32) + b1T
    hT = 0.5 * hT * (1.0 + jax.lax.erf(hT * 0.7071067811865476))
    featT = jnp.dot(w2T, hT, preferred_element_type=f32) + b2T

    dT = (jnp.dot(wfbT, featT, preferred_element_type=f32)
          + jnp.dot(ftT, oh, preferred_element_type=f32))

    varT = jnp.dot(A32[0:1, :], dT * dT, preferred_element_type=f32)
    yT = dT * jax.lax.rsqrt(varT + 1e-5) * gT + betaT
    y = yT.T
    out_ref[...] = y.reshape(out_ref.shape)


@jax.jit
def kernel(mutation_types, features, type_table, W1, b1, W2, b2, Wf, bf,
           ln_gamma, ln_beta):
    B, M = mutation_types.shape
    N = B * M
    nblk = N // TB
    mtf = mutation_types.astype(jnp.float32).reshape(nblk, 1, TB)

    small = lambda shp: pl.BlockSpec(shp, lambda i: (0,) * len(shp))
    out = pl.pallas_call(
        _body,
        grid=(nblk,),
        in_specs=[
            pl.BlockSpec((1, 1, TB), lambda i: (i, 0, 0)),
            pl.BlockSpec((TB // M, M, N_FEATURES), lambda i: (i, 0, 0)),
            small((N_TYPES, HALF)),
            small((N_FEATURES, HALF)),
            small((1, HALF)),
            small((HALF, HALF)),
            small((1, HALF)),
            small((EMBED_DIM, EMBED_DIM)),
            small((1, EMBED_DIM)),
            small((1, EMBED_DIM)),
            small((1, EMBED_DIM)),
        ],
        out_specs=pl.BlockSpec((TB // M, M, EMBED_DIM), lambda i: (i, 0, 0)),
        out_shape=jax.ShapeDtypeStruct((B, M, EMBED_DIM), jnp.float32),
    )(mtf, features, type_table, W1, b1.reshape(1, HALF), W2, b2.reshape(1, HALF),
      Wf, bf.reshape(1, EMBED_DIM), ln_gamma.reshape(1, EMBED_DIM),
      ln_beta.reshape(1, EMBED_DIM))
    return out


# speedup vs baseline: 9.5893x; 9.5653x over previous
import jax
import jax.numpy as jnp
from jax.experimental import pallas as pl

EMBED_DIM = 32
HALF = 16
N_FEATURES = 5
N_TYPES = 6


def _body(mt_ref, f_ref, tt_ref, w1_ref, b1_ref, w2_ref, b2_ref, wf_ref,
          bf_ref, g_ref, beta_ref, out_ref):
    f32 = jnp.float32
    A32 = jnp.full((EMBED_DIM, EMBED_DIM), 1.0 / EMBED_DIM, f32)

    ft = jnp.dot(tt_ref[...], wf_ref[0:HALF, :], preferred_element_type=f32)
    ft = ft + bf_ref[...]
    ft_c = ft - jnp.dot(ft, A32, preferred_element_type=f32)
    ftT = ft_c.T

    wfb = wf_ref[HALF:EMBED_DIM, :]
    wfb_c = wfb - jnp.dot(wfb, A32, preferred_element_type=f32)
    wfbT = wfb_c.T
    w23 = jnp.dot(wfbT, w2_ref[...].T, preferred_element_type=f32)
    bias_d = jnp.dot(wfbT, b2_ref[...].T, preferred_element_type=f32)

    w1T = w1_ref[...].T
    b1T = b1_ref[...].T
    gT = g_ref[...].T
    betaT = beta_ref[...].T

    m = mt_ref[0, 0, :].reshape(1, -1)
    oh = jnp.concatenate(
        [(m == k).astype(f32) for k in range(N_TYPES)], axis=0)

    fT = f_ref[...].reshape(N_FEATURES, m.shape[1])
    hT = jnp.dot(w1T, fT, preferred_element_type=f32) + b1T
    hT = 0.5 * hT * (1.0 + jax.lax.erf(hT * 0.7071067811865476))

    dT = (jnp.dot(w23, hT, preferred_element_type=f32)
          + jnp.dot(ftT, oh, preferred_element_type=f32) + bias_d)

    varT = jnp.dot(A32[0:1, :], dT * dT, preferred_element_type=f32)
    yT = dT * jax.lax.rsqrt(varT + 1e-5) * gT + betaT
    out_ref[...] = yT[None]


@jax.jit
def kernel(mutation_types, features, type_table, W1, b1, W2, b2, Wf, bf,
           ln_gamma, ln_beta):
    B, M = mutation_types.shape
    mtT = mutation_types.T.reshape(M, 1, B)
    fT = jnp.transpose(features, (2, 1, 0)).reshape(N_FEATURES, M, 1, B)

    small = lambda shp: pl.BlockSpec(shp, lambda i: (0,) * len(shp))
    out = pl.pallas_call(
        _body,
        grid=(M,),
        in_specs=[
            pl.BlockSpec((1, 1, B), lambda i: (i, 0, 0)),
            pl.BlockSpec((N_FEATURES, 1, 1, B), lambda i: (0, i, 0, 0)),
            small((N_TYPES, HALF)),
            small((N_FEATURES, HALF)),
            small((1, HALF)),
            small((HALF, HALF)),
            small((1, HALF)),
            small((EMBED_DIM, EMBED_DIM)),
            small((1, EMBED_DIM)),
            small((1, EMBED_DIM)),
            small((1, EMBED_DIM)),
        ],
        out_specs=pl.BlockSpec((1, EMBED_DIM, B), lambda i: (i, 0, 0)),
        out_shape=jax.ShapeDtypeStruct((M, EMBED_DIM, B), jnp.float32),
    )(mtT, fT, type_table, W1, b1.reshape(1, HALF), W2, b2.reshape(1, HALF),
      Wf, bf.reshape(1, EMBED_DIM), ln_gamma.reshape(1, EMBED_DIM),
      ln_beta.reshape(1, EMBED_DIM))
    return jnp.transpose(out, (2, 0, 1))


# speedup vs baseline: 11.9786x; 1.2492x over previous
import jax
import jax.numpy as jnp
from jax.experimental import pallas as pl

EMBED_DIM = 32
HALF = 16
N_FEATURES = 5
N_TYPES = 6


def _body(mt_ref, f_ref, tt_ref, w1_ref, b1_ref, w2_ref, b2_ref, wf_ref,
          bf_ref, g_ref, beta_ref, out_ref):
    f32 = jnp.float32
    i = pl.program_id(0)
    A32 = jnp.full((EMBED_DIM, EMBED_DIM), 1.0 / EMBED_DIM, f32)

    ft = jnp.dot(tt_ref[...], wf_ref[0:HALF, :], preferred_element_type=f32)
    ft = ft + bf_ref[...]
    ft_c = ft - jnp.dot(ft, A32, preferred_element_type=f32)
    ftT = ft_c.T

    wfb = wf_ref[HALF:EMBED_DIM, :]
    wfb_c = wfb - jnp.dot(wfb, A32, preferred_element_type=f32)
    wfbT = wfb_c.T
    w23 = jnp.dot(wfbT, w2_ref[...].T, preferred_element_type=f32)
    bias_d = jnp.dot(wfbT, b2_ref[...].T, preferred_element_type=f32)

    w1T = w1_ref[...].T
    b1T = b1_ref[...].T
    gT = g_ref[...].T
    betaT = beta_ref[...].T

    m = mt_ref[i, :].reshape(1, -1)
    oh = jnp.concatenate(
        [(m == k).astype(f32) for k in range(N_TYPES)], axis=0)

    fT = f_ref[:, i, :]
    hT = jnp.dot(w1T, fT, preferred_element_type=f32) + b1T
    hT = 0.5 * hT * (1.0 + jax.lax.erf(hT * 0.7071067811865476))

    dT = (jnp.dot(w23, hT, preferred_element_type=f32)
          + jnp.dot(ftT, oh, preferred_element_type=f32) + bias_d)

    varT = jnp.dot(A32[0:1, :], dT * dT, preferred_element_type=f32)
    yT = dT * jax.lax.rsqrt(varT + 1e-5) * gT + betaT
    out_ref[...] = yT[None]


@jax.jit
def kernel(mutation_types, features, type_table, W1, b1, W2, b2, Wf, bf,
           ln_gamma, ln_beta):
    B, M = mutation_types.shape
    mtT = mutation_types.T
    fT = jnp.transpose(features, (2, 1, 0))

    small = lambda shp: pl.BlockSpec(shp, lambda i: (0,) * len(shp))
    out = pl.pallas_call(
        _body,
        grid=(M,),
        in_specs=[
            small((M, B)),
            small((N_FEATURES, M, B)),
            small((N_TYPES, HALF)),
            small((N_FEATURES, HALF)),
            small((1, HALF)),
            small((HALF, HALF)),
            small((1, HALF)),
            small((EMBED_DIM, EMBED_DIM)),
            small((1, EMBED_DIM)),
            small((1, EMBED_DIM)),
            small((1, EMBED_DIM)),
        ],
        out_specs=pl.BlockSpec((1, EMBED_DIM, B), lambda i: (i, 0, 0)),
        out_shape=jax.ShapeDtypeStruct((M, EMBED_DIM, B), jnp.float32),
    )(mtT, fT, type_table, W1, b1.reshape(1, HALF), W2, b2.reshape(1, HALF),
      Wf, bf.reshape(1, EMBED_DIM), ln_gamma.reshape(1, EMBED_DIM),
      ln_beta.reshape(1, EMBED_DIM))
    return jnp.transpose(out, (2, 0, 1))


# speedup vs baseline: 12.5028x; 1.0438x over previous
import jax
import jax.numpy as jnp
from jax.experimental import pallas as pl

EMBED_DIM = 32
HALF = 16
N_FEATURES = 5
N_TYPES = 6


def _body(mt_ref, f_ref, tt_ref, w1_ref, b1_ref, w2_ref, b2_ref, wf_ref,
          bf_ref, g_ref, beta_ref, out_ref):
    f32 = jnp.float32
    i = pl.program_id(0)
    A32 = jnp.full((EMBED_DIM, EMBED_DIM), 1.0 / EMBED_DIM, f32)

    gT = g_ref[...].T
    wvar = (1.0 / EMBED_DIM) / jnp.square(g_ref[...])

    ft = jnp.dot(tt_ref[...], wf_ref[0:HALF, :], preferred_element_type=f32)
    ft = ft + bf_ref[...]
    ft_c = ft - jnp.dot(ft, A32, preferred_element_type=f32)
    ftT = ft_c.T * gT

    wfb = wf_ref[HALF:EMBED_DIM, :]
    wfb_c = wfb - jnp.dot(wfb, A32, preferred_element_type=f32)
    wfbT = wfb_c.T * gT
    w23 = jnp.dot(wfbT, w2_ref[...].T, preferred_element_type=f32)
    bias_d = jnp.dot(wfbT, b2_ref[...].T, preferred_element_type=f32)

    w1T = w1_ref[...].T
    b1T = b1_ref[...].T
    betaT = beta_ref[...].T

    m = mt_ref[i, :].reshape(1, -1)
    oh = (jax.lax.broadcasted_iota(jnp.int32, (N_TYPES, m.shape[1]), 0)
          == m).astype(f32)

    fT = f_ref[:, i, :]
    hT = jnp.dot(w1T, fT, preferred_element_type=f32) + b1T
    hT = 0.5 * hT * (1.0 + jax.lax.erf(hT * 0.7071067811865476))

    dT = (jnp.dot(w23, hT, preferred_element_type=f32)
          + jnp.dot(ftT, oh, preferred_element_type=f32) + bias_d)

    varT = jnp.dot(wvar, dT * dT, preferred_element_type=f32)
    yT = dT * jax.lax.rsqrt(varT + 1e-5) + betaT
    out_ref[...] = yT[None]


@jax.jit
def kernel(mutation_types, features, type_table, W1, b1, W2, b2, Wf, bf,
           ln_gamma, ln_beta):
    B, M = mutation_types.shape
    mtT = mutation_types.T
    fT = jnp.transpose(features, (2, 1, 0))

    small = lambda shp: pl.BlockSpec(shp, lambda i: (0,) * len(shp))
    out = pl.pallas_call(
        _body,
        grid=(M,),
        in_specs=[
            small((M, B)),
            small((N_FEATURES, M, B)),
            small((N_TYPES, HALF)),
            small((N_FEATURES, HALF)),
            small((1, HALF)),
            small((HALF, HALF)),
            small((1, HALF)),
            small((EMBED_DIM, EMBED_DIM)),
            small((1, EMBED_DIM)),
            small((1, EMBED_DIM)),
            small((1, EMBED_DIM)),
        ],
        out_specs=pl.BlockSpec((1, EMBED_DIM, B), lambda i: (i, 0, 0)),
        out_shape=jax.ShapeDtypeStruct((M, EMBED_DIM, B), jnp.float32),
    )(mtT, fT, type_table, W1, b1.reshape(1, HALF), W2, b2.reshape(1, HALF),
      Wf, bf.reshape(1, EMBED_DIM), ln_gamma.reshape(1, EMBED_DIM),
      ln_beta.reshape(1, EMBED_DIM))
    return jnp.transpose(out, (2, 0, 1))
